# baseline (speedup 1.0000x reference)
import functools

import jax
import jax.numpy as jnp
from jax import lax
from jax.experimental import pallas as pl
from jax.experimental.pallas import tpu as pltpu
from jax.experimental.pallas import tpu_sc as plsc

VOCAB = 1024
D = 64
BATCH = 4096
SEQ = 200

NC = 2
NS = 16
NW = NC * NS
BT = BATCH // NW // 4
SGRP = 2
NSTEP = SEQ // SGRP


_mesh = plsc.VectorSubcoreMesh(
    core_axis_name="c", subcore_axis_name="s", num_cores=NC, num_subcores=NS
)


@functools.partial(
    pl.kernel,
    out_type=jax.ShapeDtypeStruct((SEQ, 8, NW, 8, 128), jnp.float32),
    mesh=_mesh,
    scratch_types=[
        pltpu.VMEM((2, SGRP, 128), jnp.int32),
        pltpu.VMEM((2, SGRP, 128, D), jnp.float32),
        pltpu.VMEM((2, SGRP, 8, 8, 128), jnp.float32),
        pltpu.SemaphoreType.DMA,
        pltpu.SemaphoreType.DMA,
        pltpu.SemaphoreType.DMA,
        pltpu.SemaphoreType.DMA,
    ],
    compiler_params=pltpu.CompilerParams(
        use_tc_tiling_on_sc=False, needs_layout_passes=False
    ),
)
def _gather_kernel(table_hbm, idsT_hbm, out_hbm, idx_v, rows_v, trows_v, g0, g1, s0, s1):
    wid = lax.axis_index("s") * NC + lax.axis_index("c")
    bcol = wid * 128
    gsem = (g0, g1)
    ssem = (s0, s1)
    iota16 = lax.iota(jnp.int32, 16)

    def load_and_gather(ci, slot):
        s0_ = ci * SGRP
        pltpu.sync_copy(
            idsT_hbm.at[pl.ds(s0_, SGRP), pl.ds(bcol, 128)], idx_v.at[slot]
        )
        for k in range(SGRP):
            pltpu.make_async_copy(
                table_hbm.at[idx_v.at[slot, k]], rows_v.at[slot, k], gsem[slot]
            ).start()

    def retire_transpose_start_store(ci, slot):
        s0_ = ci * SGRP
        for k in range(SGRP):
            pltpu.make_async_copy(
                table_hbm.at[idx_v.at[slot, k]], rows_v.at[slot, k], gsem[slot]
            ).wait()
        for k in range(SGRP):
            src = rows_v.at[slot, k]
            dst = trows_v.at[slot, k]

            @pl.loop(0, 8)
            def _bg(bg):
                a = iota16 + 16 * bg
                for j in range(16):
                    mj = (iota16 + j) & 15
                    dtb = mj >> 3
                    dlj = mj & 7
                    for kk in range(4):
                        v = plsc.load_gather(src, [a, mj + 16 * kk])
                        plsc.store_scatter(dst, [dtb + 2 * kk, dlj, a], v)
        pltpu.make_async_copy(
            trows_v.at[slot],
            out_hbm.at[pl.ds(s0_, SGRP), :, wid],
            ssem[slot],
        ).start()

    def wait_store(ci, slot):
        s0_ = ci * SGRP
        pltpu.make_async_copy(
            trows_v.at[slot],
            out_hbm.at[pl.ds(s0_, SGRP), :, wid],
            ssem[slot],
        ).wait()

    load_and_gather(0, 0)
    load_and_gather(1, 1)
    retire_transpose_start_store(0, 0)
    retire_transpose_start_store(1, 1)

    @pl.loop(1, NSTEP // 2)
    def _pair(j):
        a = 2 * j
        wait_store(a - 2, 0)
        load_and_gather(a, 0)
        wait_store(a - 1, 1)
        load_and_gather(a + 1, 1)
        retire_transpose_start_store(a, 0)
        retire_transpose_start_store(a + 1, 1)

    wait_store(NSTEP - 2, 0)
    wait_store(NSTEP - 1, 1)


def kernel(ids, emb_table):
    idsT = ids.astype(jnp.int32).T
    out5 = _gather_kernel(emb_table, idsT)
    return out5.transpose(2, 4, 0, 1, 3).reshape(BATCH, SEQ, D)


# speedup vs baseline: 2.0167x; 2.0167x over previous
import functools

import jax
import jax.numpy as jnp
from jax import lax
from jax.experimental import pallas as pl
from jax.experimental.pallas import tpu as pltpu
from jax.experimental.pallas import tpu_sc as plsc

VOCAB = 1024
D = 64
BATCH = 4096
SEQ = 200

NC = 2
NS = 16
NW = NC * NS
BT = BATCH // NW // 4
SGRP = 2
NSTEP = SEQ // SGRP


_mesh = plsc.VectorSubcoreMesh(
    core_axis_name="c", subcore_axis_name="s", num_cores=NC, num_subcores=NS
)


@functools.partial(
    pl.kernel,
    out_type=jax.ShapeDtypeStruct((SEQ, 8, NW, 8, 128), jnp.float32),
    mesh=_mesh,
    scratch_types=[
        pltpu.VMEM((2, SGRP, 128), jnp.int32),
        pltpu.VMEM((2, SGRP, 128, D), jnp.float32),
        pltpu.VMEM((2, SGRP, 8, 8, 128), jnp.float32),
        pltpu.SemaphoreType.DMA,
        pltpu.SemaphoreType.DMA,
        pltpu.SemaphoreType.DMA,
        pltpu.SemaphoreType.DMA,
    ],
    compiler_params=pltpu.CompilerParams(
        use_tc_tiling_on_sc=False, needs_layout_passes=False
    ),
)
def _gather_kernel(table_hbm, idsT_hbm, out_hbm, idx_v, rows_v, trows_v, g0, g1, s0, s1):
    wid = lax.axis_index("s") * NC + lax.axis_index("c")
    bcol = wid * 128
    gsem = (g0, g1)
    ssem = (s0, s1)
    iota16 = lax.iota(jnp.int32, 16)

    def load_and_gather(ci, slot):
        s0_ = ci * SGRP
        pltpu.sync_copy(
            idsT_hbm.at[pl.ds(s0_, SGRP), pl.ds(bcol, 128)], idx_v.at[slot]
        )
        for k in range(SGRP):
            pltpu.make_async_copy(
                table_hbm.at[idx_v.at[slot, k]], rows_v.at[slot, k], gsem[slot]
            ).start()

    def retire_transpose_start_store(ci, slot):
        s0_ = ci * SGRP
        for k in range(SGRP):
            pltpu.make_async_copy(
                table_hbm.at[idx_v.at[slot, k]], rows_v.at[slot, k], gsem[slot]
            ).wait()
        for k in range(SGRP):
            src = rows_v.at[slot, k]
            dst = trows_v.at[slot, k]

            @plsc.parallel_loop(0, 128, unroll=4)
            def _u(u):
                bg = u >> 4
                j = u & 15
                a = iota16 + 16 * bg
                mj = (iota16 + j) & 15
                dtb = mj >> 3
                dlj = mj & 7
                for kk in range(4):
                    v = plsc.load_gather(src, [a, mj + 16 * kk])
                    plsc.store_scatter(dst, [dtb + 2 * kk, dlj, a], v)
        pltpu.make_async_copy(
            trows_v.at[slot],
            out_hbm.at[pl.ds(s0_, SGRP), :, wid],
            ssem[slot],
        ).start()

    def wait_store(ci, slot):
        s0_ = ci * SGRP
        pltpu.make_async_copy(
            trows_v.at[slot],
            out_hbm.at[pl.ds(s0_, SGRP), :, wid],
            ssem[slot],
        ).wait()

    load_and_gather(0, 0)
    load_and_gather(1, 1)
    retire_transpose_start_store(0, 0)
    retire_transpose_start_store(1, 1)

    @pl.loop(1, NSTEP // 2)
    def _pair(j):
        a = 2 * j
        wait_store(a - 2, 0)
        load_and_gather(a, 0)
        wait_store(a - 1, 1)
        load_and_gather(a + 1, 1)
        retire_transpose_start_store(a, 0)
        retire_transpose_start_store(a + 1, 1)

    wait_store(NSTEP - 2, 0)
    wait_store(NSTEP - 1, 1)


def kernel(ids, emb_table):
    idsT = ids.astype(jnp.int32).T
    out5 = _gather_kernel(emb_table, idsT)
    return out5.transpose(2, 4, 0, 1, 3).reshape(BATCH, SEQ, D)


# speedup vs baseline: 3.7661x; 1.8674x over previous
import functools

import jax
import jax.numpy as jnp
from jax import lax
from jax.experimental import pallas as pl
from jax.experimental.pallas import tpu as pltpu
from jax.experimental.pallas import tpu_sc as plsc

VOCAB = 1024
D = 64
BATCH = 4096
SEQ = 200

NC = 2
NS = 16
NW = NC * NS
SGRP = 2
NSTEP = SEQ // SGRP


_mesh = plsc.VectorSubcoreMesh(
    core_axis_name="c", subcore_axis_name="s", num_cores=NC, num_subcores=NS
)


@functools.partial(
    pl.kernel,
    out_type=jax.ShapeDtypeStruct((SEQ, 8, NW, 8, 128), jnp.float32),
    mesh=_mesh,
    scratch_types=[
        pltpu.VMEM((VOCAB, D), jnp.float32),
        pltpu.VMEM((2, SGRP, 128), jnp.int32),
        pltpu.VMEM((2, SGRP, 8, 8, 128), jnp.float32),
        pltpu.SemaphoreType.DMA,
        pltpu.SemaphoreType.DMA,
    ],
    compiler_params=pltpu.CompilerParams(
        use_tc_tiling_on_sc=False, needs_layout_passes=False
    ),
)
def _gather_kernel(table_hbm, idsT_hbm, out_hbm, table_v, idx_v, trows_v, s0, s1):
    wid = lax.axis_index("s") * NC + lax.axis_index("c")
    bcol = wid * 128
    ssem = (s0, s1)
    iota16 = lax.iota(jnp.int32, 16)

    pltpu.sync_copy(table_hbm, table_v)

    def load_ids(ci, slot):
        pltpu.sync_copy(
            idsT_hbm.at[pl.ds(ci * SGRP, SGRP), pl.ds(bcol, 128)], idx_v.at[slot]
        )

    def build_slab(slot):
        for k in range(SGRP):
            ids_ref = idx_v.at[slot, k]
            dst = trows_v.at[slot, k]

            @plsc.parallel_loop(0, 128, unroll=2)
            def _u(u):
                bg = u >> 4
                j = u & 15
                iv = ids_ref[pl.ds(16 * bg, 16)]
                a = iota16 + 16 * bg
                mj = (iota16 + j) & 15
                dtb = mj >> 3
                dlj = mj & 7
                for kk in range(4):
                    v = plsc.load_gather(table_v, [iv, mj + 16 * kk])
                    plsc.store_scatter(dst, [dtb + 2 * kk, dlj, a], v)

    def start_store(ci, slot):
        pltpu.make_async_copy(
            trows_v.at[slot],
            out_hbm.at[pl.ds(ci * SGRP, SGRP), :, wid],
            ssem[slot],
        ).start()

    def wait_store(ci, slot):
        pltpu.make_async_copy(
            trows_v.at[slot],
            out_hbm.at[pl.ds(ci * SGRP, SGRP), :, wid],
            ssem[slot],
        ).wait()

    load_ids(0, 0)
    build_slab(0)
    start_store(0, 0)
    load_ids(1, 1)
    build_slab(1)
    start_store(1, 1)

    @pl.loop(1, NSTEP // 2)
    def _pair(j):
        a = 2 * j
        wait_store(a - 2, 0)
        load_ids(a, 0)
        build_slab(0)
        start_store(a, 0)
        wait_store(a - 1, 1)
        load_ids(a + 1, 1)
        build_slab(1)
        start_store(a + 1, 1)

    wait_store(NSTEP - 2, 0)
    wait_store(NSTEP - 1, 1)


def kernel(ids, emb_table):
    idsT = ids.astype(jnp.int32).T
    out5 = _gather_kernel(emb_table, idsT)
    return out5.transpose(2, 4, 0, 1, 3).reshape(BATCH, SEQ, D)


# speedup vs baseline: 3.8680x; 1.0270x over previous
import functools

import jax
import jax.numpy as jnp
from jax import lax
from jax.experimental import pallas as pl
from jax.experimental.pallas import tpu as pltpu
from jax.experimental.pallas import tpu_sc as plsc

VOCAB = 1024
D = 64
BATCH = 4096
SEQ = 200

NC = 2
NS = 16
NW = NC * NS
SGRP = 2
NSTEP = SEQ // SGRP


_mesh = plsc.VectorSubcoreMesh(
    core_axis_name="c", subcore_axis_name="s", num_cores=NC, num_subcores=NS
)


@functools.partial(
    pl.kernel,
    out_type=jax.ShapeDtypeStruct((SEQ, 8, NW, 8, 128), jnp.float32),
    mesh=_mesh,
    scratch_types=[
        pltpu.VMEM((VOCAB, D), jnp.float32),
        pltpu.VMEM((2, SGRP, 128), jnp.int32),
        pltpu.VMEM((2, SGRP, 8, 8, 128), jnp.float32),
        pltpu.SemaphoreType.DMA,
        pltpu.SemaphoreType.DMA,
    ],
    compiler_params=pltpu.CompilerParams(
        use_tc_tiling_on_sc=False, needs_layout_passes=False
    ),
)
def _gather_kernel(table_hbm, idsT_hbm, out_hbm, table_v, idx_v, trows_v, s0, s1):
    wid = lax.axis_index("s") * NC + lax.axis_index("c")
    bcol = wid * 128
    ssem = (s0, s1)
    iota16 = lax.iota(jnp.int32, 16)

    pltpu.sync_copy(table_hbm, table_v)

    def load_ids(ci, slot):
        pltpu.sync_copy(
            idsT_hbm.at[pl.ds(ci * SGRP, SGRP), pl.ds(bcol, 128)], idx_v.at[slot]
        )

    def build_slab(slot):
        for k in range(SGRP):
            ids_ref = idx_v.at[slot, k]
            dst = trows_v.at[slot, k]

            @plsc.parallel_loop(0, 128, unroll=4)
            def _u(u):
                bg = u >> 4
                j = u & 15
                iv = ids_ref[pl.ds(16 * bg, 16)]
                a = iota16 + 16 * bg
                mj = (iota16 + j) & 15
                dtb = mj >> 3
                dlj = mj & 7
                for kk in range(4):
                    v = plsc.load_gather(table_v, [iv, mj + 16 * kk])
                    plsc.store_scatter(dst, [dtb + 2 * kk, dlj, a], v)

    def start_store(ci, slot):
        pltpu.make_async_copy(
            trows_v.at[slot],
            out_hbm.at[pl.ds(ci * SGRP, SGRP), :, wid],
            ssem[slot],
        ).start()

    def wait_store(ci, slot):
        pltpu.make_async_copy(
            trows_v.at[slot],
            out_hbm.at[pl.ds(ci * SGRP, SGRP), :, wid],
            ssem[slot],
        ).wait()

    load_ids(0, 0)
    build_slab(0)
    start_store(0, 0)
    load_ids(1, 1)
    build_slab(1)
    start_store(1, 1)

    @pl.loop(1, NSTEP // 2)
    def _pair(j):
        a = 2 * j
        wait_store(a - 2, 0)
        load_ids(a, 0)
        build_slab(0)
        start_store(a, 0)
        wait_store(a - 1, 1)
        load_ids(a + 1, 1)
        build_slab(1)
        start_store(a + 1, 1)

    wait_store(NSTEP - 2, 0)
    wait_store(NSTEP - 1, 1)


def kernel(ids, emb_table):
    idsT = ids.astype(jnp.int32).T
    out5 = _gather_kernel(emb_table, idsT)
    return out5.transpose(2, 4, 0, 1, 3).reshape(BATCH, SEQ, D)


# speedup vs baseline: 5.6688x; 1.4656x over previous
import functools

import jax
import jax.numpy as jnp
from jax import lax
from jax.experimental import pallas as pl
from jax.experimental.pallas import tpu as pltpu
from jax.experimental.pallas import tpu_sc as plsc

VOCAB = 1024
D = 64
BATCH = 4096
SEQ = 200

NC = 2
NS = 16
NW = NC * NS
SGRP = 2
NSTEP = SEQ // SGRP


_mesh = plsc.VectorSubcoreMesh(
    core_axis_name="c", subcore_axis_name="s", num_cores=NC, num_subcores=NS
)


@functools.partial(
    pl.kernel,
    out_type=jax.ShapeDtypeStruct((SEQ, 8, NW, 8, 128), jnp.float32),
    mesh=_mesh,
    scratch_types=[
        pltpu.VMEM((VOCAB, D), jnp.float32),
        pltpu.VMEM((SEQ, 128), jnp.int32),
        pltpu.VMEM((2, SGRP, 8, 8, 128), jnp.float32),
        pltpu.SemaphoreType.DMA,
        pltpu.SemaphoreType.DMA,
    ],
    compiler_params=pltpu.CompilerParams(
        use_tc_tiling_on_sc=False, needs_layout_passes=False
    ),
)
def _gather_kernel(table_hbm, idsT_hbm, out_hbm, table_v, idx_v, trows_v, s0, s1):
    wid = lax.axis_index("s") * NC + lax.axis_index("c")
    bcol = wid * 128
    ssem = (s0, s1)
    iota16 = lax.iota(jnp.int32, 16)

    pltpu.sync_copy(table_hbm, table_v)
    pltpu.sync_copy(idsT_hbm.at[:, pl.ds(bcol, 128)], idx_v)

    def build_slab(ci, slot):
        for k in range(SGRP):
            ids_ref = idx_v.at[ci * SGRP + k]
            dst = trows_v.at[slot, k]

            @plsc.parallel_loop(0, 128, unroll=4)
            def _u(u):
                bg = u >> 4
                j = u & 15
                iv = ids_ref[pl.ds(16 * bg, 16)]
                a = iota16 + 16 * bg
                mj = (iota16 + j) & 15
                dtb = mj >> 3
                dlj = mj & 7
                for kk in range(4):
                    v = plsc.load_gather(table_v, [iv, mj + 16 * kk])
                    plsc.store_scatter(dst, [dtb + 2 * kk, dlj, a], v)

    def start_store(ci, slot):
        pltpu.make_async_copy(
            trows_v.at[slot],
            out_hbm.at[pl.ds(ci * SGRP, SGRP), :, wid],
            ssem[slot],
        ).start()

    def wait_store(ci, slot):
        pltpu.make_async_copy(
            trows_v.at[slot],
            out_hbm.at[pl.ds(ci * SGRP, SGRP), :, wid],
            ssem[slot],
        ).wait()

    build_slab(0, 0)
    start_store(0, 0)
    build_slab(1, 1)
    start_store(1, 1)

    @pl.loop(1, NSTEP // 2)
    def _pair(j):
        a = 2 * j
        wait_store(a - 2, 0)
        build_slab(a, 0)
        start_store(a, 0)
        wait_store(a - 1, 1)
        build_slab(a + 1, 1)
        start_store(a + 1, 1)

    wait_store(NSTEP - 2, 0)
    wait_store(NSTEP - 1, 1)


def kernel(ids, emb_table):
    idsT = ids.astype(jnp.int32).T
    out5 = _gather_kernel(emb_table, idsT)
    return out5.transpose(2, 4, 0, 1, 3).reshape(BATCH, SEQ, D)
